# baseline (speedup 1.0000x reference)
import functools

import jax
import jax.numpy as jnp
from jax.experimental import pallas as pl

_DN_NN = (((1,), (0,)), ((), ()))
_DN_NT = (((1,), (1,)), ((), ()))
_DN_TN = (((0,), (0,)), ((), ()))


def _bdot(a, b, dnums):
    return jax.lax.dot_general(a.astype(jnp.bfloat16), b.astype(jnp.bfloat16),
                               dnums, preferred_element_type=jnp.float32)


def _gate(h_ref, p_ref):
    hb = h_ref[...].astype(jnp.bfloat16).astype(jnp.float32)
    pb = p_ref[...].astype(jnp.bfloat16).astype(jnp.float32)
    return jnp.sum(hb * pb, axis=1)


def _node_layer_kernel(hv_ref, he_ref, adjr_ref, tr_ref, tb_ref, w_ref,
                       b_ref, p_ref, out_ref, *, block_rows):
    i = pl.program_id(0)
    d = _gate(he_ref, p_ref)
    s = tr_ref[...] * d[None, :]
    mult = _bdot(s, tb_ref[...], _DN_NT)
    br, nv = mult.shape
    row = jax.lax.broadcasted_iota(jnp.int32, (br, nv), 0)
    col = jax.lax.broadcasted_iota(jnp.int32, (br, nv), 1)
    m = jnp.where(col == row + i * block_rows, 1.0, mult)
    adjusted = m * adjr_ref[...]
    hw = _bdot(hv_ref[...], w_ref[...], _DN_NN)
    out_ref[...] = _bdot(adjusted, hw, _DN_NN) + b_ref[...]


def _edge_layer_kernel(hv_ref, he_ref, adjr_ref, tc_ref, tb_ref, w_ref,
                       b_ref, p_ref, out_ref, *, block_rows):
    i = pl.program_id(0)
    d = _gate(hv_ref, p_ref)
    s = tc_ref[...] * d[:, None]
    mult = _bdot(s, tb_ref[...], _DN_TN)
    br, ne = mult.shape
    row = jax.lax.broadcasted_iota(jnp.int32, (br, ne), 0)
    col = jax.lax.broadcasted_iota(jnp.int32, (br, ne), 1)
    m = jnp.where(col == row + i * block_rows, 1.0, mult)
    adjusted = m * adjr_ref[...]
    hw = _bdot(he_ref[...], w_ref[...], _DN_NN)
    out_ref[...] = _bdot(adjusted, hw, _DN_NN) + b_ref[...]


def _node_layer(hv, he, adj_v, t, tb, w, b, p, block_rows=256):
    n_v, n_e = t.shape
    f_in, f_out = w.shape
    f_e = he.shape[1]
    grid = n_v // block_rows
    return pl.pallas_call(
        functools.partial(_node_layer_kernel, block_rows=block_rows),
        grid=(grid,),
        in_specs=[
            pl.BlockSpec((n_v, f_in), lambda i: (0, 0)),
            pl.BlockSpec((n_e, f_e), lambda i: (0, 0)),
            pl.BlockSpec((block_rows, n_v), lambda i: (i, 0)),
            pl.BlockSpec((block_rows, n_e), lambda i: (i, 0)),
            pl.BlockSpec((n_v, n_e), lambda i: (0, 0)),
            pl.BlockSpec((f_in, f_out), lambda i: (0, 0)),
            pl.BlockSpec((1, f_out), lambda i: (0, 0)),
            pl.BlockSpec((1, f_e), lambda i: (0, 0)),
        ],
        out_specs=pl.BlockSpec((block_rows, f_out), lambda i: (i, 0)),
        out_shape=jax.ShapeDtypeStruct((n_v, f_out), jnp.float32),
    )(hv, he, adj_v, t, tb, w, b.reshape(1, -1), p)


def _edge_layer(hv, he, adj_e, t, tb, w, b, p, block_rows=256):
    n_v, n_e = t.shape
    f_in, f_out = w.shape
    f_v = hv.shape[1]
    grid = n_e // block_rows
    return pl.pallas_call(
        functools.partial(_edge_layer_kernel, block_rows=block_rows),
        grid=(grid,),
        in_specs=[
            pl.BlockSpec((n_v, f_v), lambda i: (0, 0)),
            pl.BlockSpec((n_e, f_in), lambda i: (0, 0)),
            pl.BlockSpec((block_rows, n_e), lambda i: (i, 0)),
            pl.BlockSpec((n_v, block_rows), lambda i: (0, i)),
            pl.BlockSpec((n_v, n_e), lambda i: (0, 0)),
            pl.BlockSpec((f_in, f_out), lambda i: (0, 0)),
            pl.BlockSpec((1, f_out), lambda i: (0, 0)),
            pl.BlockSpec((1, f_v), lambda i: (0, 0)),
        ],
        out_specs=pl.BlockSpec((block_rows, f_out), lambda i: (i, 0)),
        out_shape=jax.ShapeDtypeStruct((n_e, f_out), jnp.float32),
    )(hv, he, adj_e, t, tb, w, b.reshape(1, -1), p)


def kernel(X, Z, adj_e, adj_v, T, W1, b1, p1, W2, b2, p2, W3, b3, p3):
    tb = T.astype(jnp.bfloat16)
    X1 = _node_layer(X, Z, adj_v, T, tb, W1, b1, p1)
    Z2 = _edge_layer(X1, Z, adj_e, T, tb, W2, b2, p2)
    X3 = _node_layer(X1, Z2, adj_v, T, tb, W3, b3, p3)
    return X3


# speedup vs baseline: 1.1310x; 1.1310x over previous
import functools

import jax
import jax.numpy as jnp
from jax.experimental import pallas as pl
from jax.experimental.pallas import tpu as pltpu

_DN_NN = (((1,), (0,)), ((), ()))
_DN_NT = (((1,), (1,)), ((), ()))
_DN_TN = (((0,), (0,)), ((), ()))


def _bdot(a, b, dnums):
    return jax.lax.dot_general(a.astype(jnp.bfloat16), b.astype(jnp.bfloat16),
                               dnums, preferred_element_type=jnp.float32)


def _gate(h_ref, p_ref):
    hb = h_ref[...].astype(jnp.bfloat16).astype(jnp.float32)
    pb = p_ref[...].astype(jnp.bfloat16).astype(jnp.float32)
    return jnp.sum(hb * pb, axis=1)


def _node_layer_kernel(hv_ref, he_ref, adjr_ref, tb_ref, w_ref, b_ref, p_ref,
                       out_ref, d_scr, hw_scr, *, block_rows):
    i = pl.program_id(0)

    @pl.when(i == 0)
    def _prologue():
        d_scr[...] = _gate(he_ref, p_ref)[None, :]
        hw_scr[...] = _bdot(hv_ref[...], w_ref[...], _DN_NN).astype(jnp.bfloat16)

    trow = tb_ref[pl.ds(i * block_rows, block_rows), :].astype(jnp.float32)
    s = trow * d_scr[...]
    mult = _bdot(s, tb_ref[...], _DN_NT)
    br, nv = mult.shape
    row = jax.lax.broadcasted_iota(jnp.int32, (br, nv), 0)
    col = jax.lax.broadcasted_iota(jnp.int32, (br, nv), 1)
    m = jnp.where(col == row + i * block_rows, 1.0, mult)
    adjusted = m * adjr_ref[...]
    out_ref[...] = _bdot(adjusted, hw_scr[...], _DN_NN) + b_ref[...]


def _edge_layer_kernel(hv_ref, he_ref, adjr_ref, tb_ref, w_ref, b_ref, p_ref,
                       out_ref, d_scr, hw_scr, *, block_rows):
    i = pl.program_id(0)

    @pl.when(i == 0)
    def _prologue():
        d_scr[...] = _gate(hv_ref, p_ref)[:, None]
        hw_scr[...] = _bdot(he_ref[...], w_ref[...], _DN_NN).astype(jnp.bfloat16)

    tcol = tb_ref[:, pl.ds(i * block_rows, block_rows)].astype(jnp.float32)
    s = tcol * d_scr[...]
    mult = _bdot(s, tb_ref[...], _DN_TN)
    br, ne = mult.shape
    row = jax.lax.broadcasted_iota(jnp.int32, (br, ne), 0)
    col = jax.lax.broadcasted_iota(jnp.int32, (br, ne), 1)
    m = jnp.where(col == row + i * block_rows, 1.0, mult)
    adjusted = m * adjr_ref[...]
    out_ref[...] = _bdot(adjusted, hw_scr[...], _DN_NN) + b_ref[...]


def _node_layer(hv, he, adj_v, tb, w, b, p, block_rows=512):
    n_v, n_e = tb.shape
    f_in, f_out = w.shape
    f_e = he.shape[1]
    grid = n_v // block_rows
    return pl.pallas_call(
        functools.partial(_node_layer_kernel, block_rows=block_rows),
        grid=(grid,),
        in_specs=[
            pl.BlockSpec((n_v, f_in), lambda i: (0, 0)),
            pl.BlockSpec((n_e, f_e), lambda i: (0, 0)),
            pl.BlockSpec((block_rows, n_v), lambda i: (i, 0)),
            pl.BlockSpec((n_v, n_e), lambda i: (0, 0)),
            pl.BlockSpec((f_in, f_out), lambda i: (0, 0)),
            pl.BlockSpec((1, f_out), lambda i: (0, 0)),
            pl.BlockSpec((1, f_e), lambda i: (0, 0)),
        ],
        out_specs=pl.BlockSpec((block_rows, f_out), lambda i: (i, 0)),
        out_shape=jax.ShapeDtypeStruct((n_v, f_out), jnp.float32),
        scratch_shapes=[
            pltpu.VMEM((1, n_e), jnp.float32),
            pltpu.VMEM((n_v, f_out), jnp.bfloat16),
        ],
    )(hv, he, adj_v, tb, w, b.reshape(1, -1), p)


def _edge_layer(hv, he, adj_e, tb, w, b, p, block_rows=512):
    n_v, n_e = tb.shape
    f_in, f_out = w.shape
    f_v = hv.shape[1]
    grid = n_e // block_rows
    return pl.pallas_call(
        functools.partial(_edge_layer_kernel, block_rows=block_rows),
        grid=(grid,),
        in_specs=[
            pl.BlockSpec((n_v, f_v), lambda i: (0, 0)),
            pl.BlockSpec((n_e, f_in), lambda i: (0, 0)),
            pl.BlockSpec((block_rows, n_e), lambda i: (i, 0)),
            pl.BlockSpec((n_v, n_e), lambda i: (0, 0)),
            pl.BlockSpec((f_in, f_out), lambda i: (0, 0)),
            pl.BlockSpec((1, f_out), lambda i: (0, 0)),
            pl.BlockSpec((1, f_v), lambda i: (0, 0)),
        ],
        out_specs=pl.BlockSpec((block_rows, f_out), lambda i: (i, 0)),
        out_shape=jax.ShapeDtypeStruct((n_e, f_out), jnp.float32),
        scratch_shapes=[
            pltpu.VMEM((n_v, 1), jnp.float32),
            pltpu.VMEM((n_e, f_out), jnp.bfloat16),
        ],
    )(hv, he, adj_e, tb, w, b.reshape(1, -1), p)


def kernel(X, Z, adj_e, adj_v, T, W1, b1, p1, W2, b2, p2, W3, b3, p3):
    tb = T.astype(jnp.bfloat16)
    X1 = _node_layer(X, Z, adj_v, tb, W1, b1, p1)
    Z2 = _edge_layer(X1, Z, adj_e, tb, W2, b2, p2)
    X3 = _node_layer(X1, Z2, adj_v, tb, W3, b3, p3)
    return X3
